# baseline (speedup 1.0000x reference)
import functools

import jax
import jax.numpy as jnp
from jax import lax
from jax.experimental import pallas as pl
from jax.experimental.pallas import tpu as pltpu
from jax.experimental.pallas import tpu_sc as plsc

_E = 16
_CAP = 1024
_L = 128
_LO_BITS = 0x2F03F0FF
_HI_BITS = 0x3F800000


def _probs_body(h_ref, wt_ref, p_ref, bt_ref):
    x = h_ref[...]
    wt = wt_ref[...]
    logits = jnp.dot(x, wt, preferred_element_type=jnp.float32)
    logits = jnp.clip(logits, -10.0, 10.0)
    m = jnp.max(logits, axis=-1, keepdims=True)
    e = jnp.exp(logits - m)
    p = e / jnp.sum(e, axis=-1, keepdims=True)
    p_ref[...] = p
    bt_ref[...] = lax.bitcast_convert_type(jnp.transpose(p), jnp.int32)


def _sc_select_body(bt_hbm, t_hbm, need_hbm, bits_v, hist_v, stage_v):
    wid = lax.axis_index("s") * 2 + lax.axis_index("c")

    @pl.when(wid < _E)
    def _():
        n = bits_v.shape[0]
        pltpu.sync_copy(bt_hbm.at[wid], bits_v)

        ones = jnp.ones((16,), jnp.int32)
        zeros = jnp.zeros((16,), jnp.int32)
        path = jnp.zeros((16,), jnp.int32)
        rank = jnp.full((16,), _CAP, jnp.int32)

        for shift in (25, 20, 15, 10, 5, 0):
            for c in range(16):
                hist_v[pl.ds(c * 16, 16)] = zeros

            def build(i, carry, shift=shift, path=path):
                for u in range(8):
                    v = bits_v[pl.ds(i * 128 + u * 16, 16)]
                    q = v - _LO_BITS
                    m = (q >> (shift + 5)) == path
                    d = ((q >> shift) & 31) + u * 32
                    plsc.addupdate_scatter(hist_v, [d], ones, mask=m)
                return carry

            lax.fori_loop(0, n // 128, build, 0)

            v0 = hist_v[pl.ds(0, 16)]
            v1 = hist_v[pl.ds(16, 16)]
            for c in range(1, 8):
                v0 = v0 + hist_v[pl.ds(c * 32, 16)]
                v1 = v1 + hist_v[pl.ds(c * 32 + 16, 16)]
            t1 = jnp.sum(v1)
            sfx0 = lax.rev(plsc.cumsum(lax.rev(v0, (0,))), (0,)) + t1
            sfx1 = lax.rev(plsc.cumsum(lax.rev(v1, (0,))), (0,))
            m0 = sfx0 >= rank
            m1 = sfx1 >= rank
            pc = (plsc.all_reduce_population_count(m0) +
                  plsc.all_reduce_population_count(m1))
            bstar = pc - 1
            above = (jnp.sum(jnp.where(m0, 0, v0)) +
                     jnp.sum(jnp.where(m1, 0, v1)))
            rank = rank - above
            path = path * 32 + bstar

        stage_v[pl.ds(0, 16)] = path + _LO_BITS
        stage_v[pl.ds(16, 16)] = rank
        pltpu.sync_copy(stage_v.at[pl.ds(0, 16)], t_hbm.at[pl.ds(wid * 16, 16)])
        pltpu.sync_copy(stage_v.at[pl.ds(16, 16)],
                        need_hbm.at[pl.ds(wid * 16, 16)])


def _finalize_body(p_ref, t_ref, need_ref, w_ref, m_ref, bits_ref, icut_ref):
    rows, lanes = p_ref.shape
    n = rows * lanes // _E
    bits_ref[...] = lax.bitcast_convert_type(p_ref[...], jnp.int32)

    li = lax.broadcasted_iota(jnp.int32, (lanes, lanes), 0)
    lj = lax.broadcasted_iota(jnp.int32, (lanes, lanes), 1)
    m_exp = ((li & (_E - 1)) == (lj & (_E - 1))).astype(jnp.float32)
    m_tok = ((li // _E) == (lj // _E)).astype(jnp.float32)

    ei = lax.broadcasted_iota(jnp.int32, (_E, _E), 0)
    ej = lax.broadcasted_iota(jnp.int32, (_E, _E), 1)
    eye = ei == ej
    t16 = jnp.sum(jnp.where(eye, t_ref[...], 0), axis=0, keepdims=True)
    need16 = jnp.sum(jnp.where(eye, need_ref[...], 0), axis=0, keepdims=True)
    tbits = jnp.concatenate([t16] * (lanes // _E), axis=1)
    needf = jnp.concatenate([need16] * (lanes // _E), axis=1).astype(
        jnp.float32)

    capf = float(_CAP)

    def count_exp(x_bool):
        s = jnp.sum(x_bool.astype(jnp.float32), axis=0, keepdims=True)
        return jnp.dot(s, m_exp, preferred_element_type=jnp.float32)

    idx = (lax.broadcasted_iota(jnp.int32, (rows, lanes), 0) * (lanes // _E) +
           (lax.broadcasted_iota(jnp.int32, (rows, lanes), 1) // _E))

    icut_ref[...] = jnp.full((1, lanes), n - 1, jnp.int32)
    cge = count_exp(bits_ref[...] >= tbits)
    has_ties = jnp.any(cge > capf)

    @pl.when(has_ties)
    def _():
        def idx_body(_, carry):
            lo, hi = carry
            mid = (lo + hi) >> 1
            ok = count_exp((bits_ref[...] == tbits) &
                           (idx <= mid)) >= needf
            return jnp.where(ok, lo, mid + 1), jnp.where(ok, mid, hi)

        ilo0 = jnp.zeros((1, lanes), jnp.int32)
        ihi0 = jnp.full((1, lanes), n - 1, jnp.int32)
        res, _ = lax.fori_loop(0, 13, idx_body, (ilo0, ihi0))
        icut_ref[...] = res

    icut = icut_ref[...]

    p = p_ref[...]
    bits = bits_ref[...]
    mask = (bits > tbits) | ((bits == tbits) & (idx <= icut))
    maskf = mask.astype(jnp.float32)
    wun = maskf * p
    denom = jnp.dot(wun, m_tok, preferred_element_type=jnp.float32) + 1e-10
    w_ref[...] = wun / denom
    m_ref[...] = maskf


def kernel(hidden_states, gate_weight):
    b, s, d = hidden_states.shape
    n = b * s
    h = hidden_states.reshape(n, d)
    wt = gate_weight.T

    tok_blk = 512
    probs, bits_t = pl.pallas_call(
        _probs_body,
        grid=(n // tok_blk,),
        in_specs=[
            pl.BlockSpec((tok_blk, d), lambda i: (i, 0)),
            pl.BlockSpec((d, _E), lambda i: (0, 0)),
        ],
        out_specs=[
            pl.BlockSpec((tok_blk, _E), lambda i: (i, 0)),
            pl.BlockSpec((_E, tok_blk), lambda i: (0, i)),
        ],
        out_shape=[
            jax.ShapeDtypeStruct((n, _E), jnp.float32),
            jax.ShapeDtypeStruct((_E, n), jnp.int32),
        ],
    )(h, wt)

    mesh = plsc.VectorSubcoreMesh(core_axis_name="c", subcore_axis_name="s")
    sc_select = functools.partial(
        pl.kernel,
        mesh=mesh,
        out_type=[
            jax.ShapeDtypeStruct((_E * 16,), jnp.int32),
            jax.ShapeDtypeStruct((_E * 16,), jnp.int32),
        ],
        scratch_types=[
            pltpu.VMEM((n,), jnp.int32),
            pltpu.VMEM((256,), jnp.int32),
            pltpu.VMEM((32,), jnp.int32),
        ],
        compiler_params=pltpu.CompilerParams(needs_layout_passes=False),
    )(_sc_select_body)
    tvec, needvec = sc_select(bits_t)
    tmat = tvec.reshape(_E, 16)
    needmat = needvec.reshape(_E, 16)

    rows = n * _E // _L
    probs_packed = probs.reshape(rows, _L)

    w, m = pl.pallas_call(
        _finalize_body,
        in_specs=[
            pl.BlockSpec((rows, _L), lambda: (0, 0)),
            pl.BlockSpec((_E, 16), lambda: (0, 0)),
            pl.BlockSpec((_E, 16), lambda: (0, 0)),
        ],
        out_specs=[
            pl.BlockSpec((rows, _L), lambda: (0, 0)),
            pl.BlockSpec((rows, _L), lambda: (0, 0)),
        ],
        out_shape=[
            jax.ShapeDtypeStruct((rows, _L), jnp.float32),
            jax.ShapeDtypeStruct((rows, _L), jnp.float32),
        ],
        scratch_shapes=[
            pltpu.VMEM((rows, _L), jnp.int32),
            pltpu.VMEM((1, _L), jnp.int32),
        ],
    )(probs_packed, tmat, needmat)

    return w.reshape(b, s, _E), m.reshape(b, s, _E)


# speedup vs baseline: 1.2911x; 1.2911x over previous
import functools

import jax
import jax.numpy as jnp
from jax import lax
from jax.experimental import pallas as pl
from jax.experimental.pallas import tpu as pltpu
from jax.experimental.pallas import tpu_sc as plsc

_E = 16
_CAP = 1024
_L = 128
_LO_BITS = 0x2F03F0FF
_HI_BITS = 0x3F800000


def _probs_body(h_ref, wt_ref, p_ref, bt_ref):
    x = h_ref[...]
    wt = wt_ref[...]
    logits = jnp.dot(x, wt, preferred_element_type=jnp.float32)
    logits = jnp.clip(logits, -10.0, 10.0)
    m = jnp.max(logits, axis=-1, keepdims=True)
    e = jnp.exp(logits - m)
    p = e / jnp.sum(e, axis=-1, keepdims=True)
    p_ref[...] = p
    bt_ref[...] = lax.bitcast_convert_type(jnp.transpose(p), jnp.int32)


def _sc_select_body(bt_hbm, t_hbm, need_hbm, bits_v, hist_v, stage_v):
    wid = lax.axis_index("s") * 2 + lax.axis_index("c")

    @pl.when(wid < _E)
    def _():
        n = bits_v.shape[0]
        pltpu.sync_copy(bt_hbm.at[wid], bits_v)

        ones = jnp.ones((16,), jnp.int32)
        zeros = jnp.zeros((16,), jnp.int32)
        path = jnp.zeros((16,), jnp.int32)
        rank = jnp.full((16,), _CAP, jnp.int32)

        for shift in (25, 20, 15, 10, 5, 0):
            for c in range(16):
                hist_v[pl.ds(c * 16, 16)] = zeros

            @plsc.parallel_loop(0, n // 16, 1, unroll=8)
            def build(i, shift=shift, path=path):
                v = bits_v[pl.ds(i * 16, 16)]
                q = v - _LO_BITS
                m = (q >> (shift + 5)) == path
                d = ((q >> shift) & 31) + (i & 7) * 32
                plsc.addupdate_scatter(hist_v, [d], ones, mask=m)

            v0 = hist_v[pl.ds(0, 16)]
            v1 = hist_v[pl.ds(16, 16)]
            for c in range(1, 8):
                v0 = v0 + hist_v[pl.ds(c * 32, 16)]
                v1 = v1 + hist_v[pl.ds(c * 32 + 16, 16)]
            t1 = jnp.sum(v1)
            sfx0 = lax.rev(plsc.cumsum(lax.rev(v0, (0,))), (0,)) + t1
            sfx1 = lax.rev(plsc.cumsum(lax.rev(v1, (0,))), (0,))
            m0 = sfx0 >= rank
            m1 = sfx1 >= rank
            pc = (plsc.all_reduce_population_count(m0) +
                  plsc.all_reduce_population_count(m1))
            bstar = pc - 1
            above = (jnp.sum(jnp.where(m0, 0, v0)) +
                     jnp.sum(jnp.where(m1, 0, v1)))
            rank = rank - above
            path = path * 32 + bstar

        stage_v[pl.ds(0, 16)] = path + _LO_BITS
        stage_v[pl.ds(16, 16)] = rank
        pltpu.sync_copy(stage_v.at[pl.ds(0, 16)], t_hbm.at[pl.ds(wid * 16, 16)])
        pltpu.sync_copy(stage_v.at[pl.ds(16, 16)],
                        need_hbm.at[pl.ds(wid * 16, 16)])


def _finalize_body(p_ref, t_ref, need_ref, w_ref, m_ref, bits_ref, icut_ref):
    rows, lanes = p_ref.shape
    n = rows * lanes // _E
    bits_ref[...] = lax.bitcast_convert_type(p_ref[...], jnp.int32)

    li = lax.broadcasted_iota(jnp.int32, (lanes, lanes), 0)
    lj = lax.broadcasted_iota(jnp.int32, (lanes, lanes), 1)
    m_exp = ((li & (_E - 1)) == (lj & (_E - 1))).astype(jnp.float32)
    m_tok = ((li // _E) == (lj // _E)).astype(jnp.float32)

    ei = lax.broadcasted_iota(jnp.int32, (_E, _E), 0)
    ej = lax.broadcasted_iota(jnp.int32, (_E, _E), 1)
    eye = ei == ej
    t16 = jnp.sum(jnp.where(eye, t_ref[...], 0), axis=0, keepdims=True)
    need16 = jnp.sum(jnp.where(eye, need_ref[...], 0), axis=0, keepdims=True)
    tbits = jnp.concatenate([t16] * (lanes // _E), axis=1)
    needf = jnp.concatenate([need16] * (lanes // _E), axis=1).astype(
        jnp.float32)

    capf = float(_CAP)

    def count_exp(x_bool):
        s = jnp.sum(x_bool.astype(jnp.float32), axis=0, keepdims=True)
        return jnp.dot(s, m_exp, preferred_element_type=jnp.float32)

    idx = (lax.broadcasted_iota(jnp.int32, (rows, lanes), 0) * (lanes // _E) +
           (lax.broadcasted_iota(jnp.int32, (rows, lanes), 1) // _E))

    icut_ref[...] = jnp.full((1, lanes), n - 1, jnp.int32)
    cge = count_exp(bits_ref[...] >= tbits)
    has_ties = jnp.any(cge > capf)

    @pl.when(has_ties)
    def _():
        def idx_body(_, carry):
            lo, hi = carry
            mid = (lo + hi) >> 1
            ok = count_exp((bits_ref[...] == tbits) &
                           (idx <= mid)) >= needf
            return jnp.where(ok, lo, mid + 1), jnp.where(ok, mid, hi)

        ilo0 = jnp.zeros((1, lanes), jnp.int32)
        ihi0 = jnp.full((1, lanes), n - 1, jnp.int32)
        res, _ = lax.fori_loop(0, 13, idx_body, (ilo0, ihi0))
        icut_ref[...] = res

    icut = icut_ref[...]

    p = p_ref[...]
    bits = bits_ref[...]
    mask = (bits > tbits) | ((bits == tbits) & (idx <= icut))
    maskf = mask.astype(jnp.float32)
    wun = maskf * p
    denom = jnp.dot(wun, m_tok, preferred_element_type=jnp.float32) + 1e-10
    w_ref[...] = wun / denom
    m_ref[...] = maskf


def kernel(hidden_states, gate_weight):
    b, s, d = hidden_states.shape
    n = b * s
    h = hidden_states.reshape(n, d)
    wt = gate_weight.T

    tok_blk = 512
    probs, bits_t = pl.pallas_call(
        _probs_body,
        grid=(n // tok_blk,),
        in_specs=[
            pl.BlockSpec((tok_blk, d), lambda i: (i, 0)),
            pl.BlockSpec((d, _E), lambda i: (0, 0)),
        ],
        out_specs=[
            pl.BlockSpec((tok_blk, _E), lambda i: (i, 0)),
            pl.BlockSpec((_E, tok_blk), lambda i: (0, i)),
        ],
        out_shape=[
            jax.ShapeDtypeStruct((n, _E), jnp.float32),
            jax.ShapeDtypeStruct((_E, n), jnp.int32),
        ],
    )(h, wt)

    mesh = plsc.VectorSubcoreMesh(core_axis_name="c", subcore_axis_name="s")
    sc_select = functools.partial(
        pl.kernel,
        mesh=mesh,
        out_type=[
            jax.ShapeDtypeStruct((_E * 16,), jnp.int32),
            jax.ShapeDtypeStruct((_E * 16,), jnp.int32),
        ],
        scratch_types=[
            pltpu.VMEM((n,), jnp.int32),
            pltpu.VMEM((256,), jnp.int32),
            pltpu.VMEM((32,), jnp.int32),
        ],
        compiler_params=pltpu.CompilerParams(needs_layout_passes=False),
    )(_sc_select_body)
    tvec, needvec = sc_select(bits_t)
    tmat = tvec.reshape(_E, 16)
    needmat = needvec.reshape(_E, 16)

    rows = n * _E // _L
    probs_packed = probs.reshape(rows, _L)

    w, m = pl.pallas_call(
        _finalize_body,
        in_specs=[
            pl.BlockSpec((rows, _L), lambda: (0, 0)),
            pl.BlockSpec((_E, 16), lambda: (0, 0)),
            pl.BlockSpec((_E, 16), lambda: (0, 0)),
        ],
        out_specs=[
            pl.BlockSpec((rows, _L), lambda: (0, 0)),
            pl.BlockSpec((rows, _L), lambda: (0, 0)),
        ],
        out_shape=[
            jax.ShapeDtypeStruct((rows, _L), jnp.float32),
            jax.ShapeDtypeStruct((rows, _L), jnp.float32),
        ],
        scratch_shapes=[
            pltpu.VMEM((rows, _L), jnp.int32),
            pltpu.VMEM((1, _L), jnp.int32),
        ],
    )(probs_packed, tmat, needmat)

    return w.reshape(b, s, _E), m.reshape(b, s, _E)
